# baseline (speedup 1.0000x reference)
import jax
import jax.numpy as jnp
from jax.experimental import pallas as pl
from jax.experimental.pallas import tpu as pltpu

_B, _N, _F = 16, 128, 16
_FC_OUT = 256
_BI = 8


def _r(v):
    return v.astype(jnp.bfloat16).astype(jnp.float32)


def _pe(wt_ref, t, act):
    act = _r(act)
    res = wt_ref[t, 0].astype(jnp.float32)[None, :, :] * act[:, 0, None, :]
    for g in range(1, _F):
        res = res + (wt_ref[t, g].astype(jnp.float32)[None, :, :]
                     * act[:, g, None, :])
    return res


def _xw(wt_ref, t, xi_bf):
    w2d = wt_ref[t].reshape(_F, _F * _N)
    res = jax.lax.dot_general(xi_bf, w2d, (((1,), (0,)), ((), ())),
                              preferred_element_type=jnp.float32)
    return res.reshape(_B, _F, _N)


def _gnn_kernel(x_ref, a_ref, wz_ref, wr_ref, wm_ref, uz_ref, ur_ref, um_ref,
                bz_ref, br_ref, bm_ref, un_ref, umsg_ref, lw_ref, lb_ref,
                out_ref):
    it = pl.program_id(0)

    @pl.when(it == 0)
    def _init():
        out_ref[...] = jnp.zeros_like(out_ref)

    bz = bz_ref[...][:, :, None]
    br = br_ref[...][:, :, None]
    bm = bm_ref[...][:, :, None]

    for t in range(_BI):
        xi = x_ref[t]
        af = (a_ref[t] > 0).astype(jnp.float32)
        a1 = af[:, None, :]

        xi_bf = xi.astype(jnp.bfloat16)
        xz = _xw(wz_ref, t, xi_bf) + bz
        xr = _xw(wr_ref, t, xi_bf) + br
        xm = _xw(wm_ref, t, xi_bf) + bm

        z1 = jax.nn.sigmoid(xz)
        mt1 = jnp.tanh(xm)
        m1 = a1 * (1.0 - z1) * mt1

        ns = jnp.sum(m1, axis=2, keepdims=True)
        m_prev = ns - m1
        z2 = jax.nn.sigmoid(xz + _pe(uz_ref, t, m_prev))
        r2 = jax.nn.sigmoid(xr + _pe(ur_ref, t, m1))
        rm = r2 * m1
        s = jnp.sum(rm, axis=2, keepdims=True) - rm
        mt2 = jnp.tanh(xm + _pe(um_ref, t, s))
        m2 = a1 * (z2 * m_prev + (1.0 - z2) * mt2)

        msum = jnp.sum(m2, axis=2)
        dn = jax.lax.dot_general(_r(xi), _r(un_ref[t]),
                                 (((1,), (1,)), ((), ())),
                                 preferred_element_type=jnp.float32)
        dm = jax.lax.dot_general(_r(msum), _r(umsg_ref[t]),
                                 (((1,), (1,)), ((), ())),
                                 preferred_element_type=jnp.float32)
        enc = jax.nn.relu(dn + dm)
        lw_t = lw_ref[t]
        out_ref[...] += jax.lax.dot_general(
            _r(enc), _r(lw_t), (((1,), (1,)), ((), ())),
            preferred_element_type=jnp.float32)

    @pl.when(it == (_N // _BI) - 1)
    def _fin():
        out_ref[...] = jax.nn.sigmoid(out_ref[...] + lb_ref[...])


def kernel(node_features, adjacency_matrix, batch_size,
           w_gru_update_gate_features, w_gru_forget_gate_features,
           w_gru_current_memory_message_features,
           u_gru_update_gate, u_gru_forget_gate, u_gru_current_memory_message,
           b_gru_update_gate, b_gru_forget_gate, b_gru_current_memory_message,
           u_graph_node_features, u_graph_neighbor_messages,
           linear_w, linear_b):
    del batch_size
    bz = b_gru_update_gate.reshape(1, _F)
    br = b_gru_forget_gate.reshape(1, _F)
    bm = b_gru_current_memory_message.reshape(1, _F)
    lb = linear_b.reshape(1, _FC_OUT)
    x_t = jnp.transpose(node_features, (1, 0, 2))
    a_t = jnp.transpose(adjacency_matrix, (1, 0, 2))
    lw_t = jnp.transpose(linear_w.reshape(_FC_OUT, _N, _F), (1, 0, 2))

    def _prep(w):
        return jnp.transpose(w, (0, 3, 2, 1)).astype(jnp.bfloat16)

    wz_p = _prep(w_gru_update_gate_features)
    wr_p = _prep(w_gru_forget_gate_features)
    wm_p = _prep(w_gru_current_memory_message_features)
    uz_p = _prep(u_gru_update_gate)
    ur_p = _prep(u_gru_forget_gate)
    um_p = _prep(u_gru_current_memory_message)

    grid = (_N // _BI,)
    wspec = pl.BlockSpec((_BI, _F, _F, _N), lambda i: (i, 0, 0, 0))
    sspec = pl.BlockSpec((_BI, _F, _F), lambda i: (i, 0, 0))
    full2 = lambda shape: pl.BlockSpec(shape, lambda i: (0, 0))

    return pl.pallas_call(
        _gnn_kernel,
        grid=grid,
        in_specs=[
            pl.BlockSpec((_BI, _B, _F), lambda i: (i, 0, 0)),
            pl.BlockSpec((_BI, _B, _N), lambda i: (i, 0, 0)),
            wspec, wspec, wspec, wspec, wspec, wspec,
            full2((1, _F)), full2((1, _F)), full2((1, _F)),
            sspec, sspec,
            pl.BlockSpec((_BI, _FC_OUT, _F), lambda i: (i, 0, 0)),
            full2((1, _FC_OUT)),
        ],
        out_specs=pl.BlockSpec((_B, _FC_OUT), lambda i: (0, 0)),
        out_shape=jax.ShapeDtypeStruct((_B, _FC_OUT), jnp.float32),
        compiler_params=pltpu.CompilerParams(
            dimension_semantics=("arbitrary",)),
    )(x_t, a_t, wz_p, wr_p, wm_p, uz_p, ur_p, um_p,
      bz, br, bm, u_graph_node_features, u_graph_neighbor_messages,
      lw_t, lb)


# speedup vs baseline: 1.0042x; 1.0042x over previous
import jax
import jax.numpy as jnp
from jax.experimental import pallas as pl
from jax.experimental.pallas import tpu as pltpu

_B, _N, _F = 16, 128, 16
_FC_OUT = 256
_BI = 4


def _r(v):
    return v.astype(jnp.bfloat16).astype(jnp.float32)


def _pe(wt_ref, t, act):
    act = _r(act)
    res = wt_ref[t, 0].astype(jnp.float32)[None, :, :] * act[:, 0, None, :]
    for g in range(1, _F):
        res = res + (wt_ref[t, g].astype(jnp.float32)[None, :, :]
                     * act[:, g, None, :])
    return res


def _xw(wt_ref, t, xi_bf):
    w2d = wt_ref[t].reshape(_F, _F * _N)
    res = jax.lax.dot_general(xi_bf, w2d, (((1,), (0,)), ((), ())),
                              preferred_element_type=jnp.float32)
    return res.reshape(_B, _F, _N)


def _gnn_kernel(x_ref, a_ref, wz_ref, wr_ref, wm_ref, uz_ref, ur_ref, um_ref,
                bz_ref, br_ref, bm_ref, un_ref, umsg_ref, lw_ref, lb_ref,
                out_ref):
    it = pl.program_id(0)

    @pl.when(it == 0)
    def _init():
        out_ref[...] = jnp.zeros_like(out_ref)

    bz = bz_ref[...][:, :, None]
    br = br_ref[...][:, :, None]
    bm = bm_ref[...][:, :, None]

    for t in range(_BI):
        xi = x_ref[t]
        af = (a_ref[t] > 0).astype(jnp.float32)
        a1 = af[:, None, :]

        xi_bf = xi.astype(jnp.bfloat16)
        xz = _xw(wz_ref, t, xi_bf) + bz
        xr = _xw(wr_ref, t, xi_bf) + br
        xm = _xw(wm_ref, t, xi_bf) + bm

        z1 = jax.nn.sigmoid(xz)
        mt1 = jnp.tanh(xm)
        m1 = a1 * (1.0 - z1) * mt1

        ns = jnp.sum(m1, axis=2, keepdims=True)
        m_prev = ns - m1
        z2 = jax.nn.sigmoid(xz + _pe(uz_ref, t, m_prev))
        r2 = jax.nn.sigmoid(xr + _pe(ur_ref, t, m1))
        rm = r2 * m1
        s = jnp.sum(rm, axis=2, keepdims=True) - rm
        mt2 = jnp.tanh(xm + _pe(um_ref, t, s))
        m2 = a1 * (z2 * m_prev + (1.0 - z2) * mt2)

        msum = jnp.sum(m2, axis=2)
        dn = jax.lax.dot_general(_r(xi), _r(un_ref[t]),
                                 (((1,), (1,)), ((), ())),
                                 preferred_element_type=jnp.float32)
        dm = jax.lax.dot_general(_r(msum), _r(umsg_ref[t]),
                                 (((1,), (1,)), ((), ())),
                                 preferred_element_type=jnp.float32)
        enc = jax.nn.relu(dn + dm)
        lw_t = lw_ref[t]
        out_ref[...] += jax.lax.dot_general(
            _r(enc), _r(lw_t), (((1,), (1,)), ((), ())),
            preferred_element_type=jnp.float32)

    @pl.when(it == (_N // _BI) - 1)
    def _fin():
        out_ref[...] = jax.nn.sigmoid(out_ref[...] + lb_ref[...])


def kernel(node_features, adjacency_matrix, batch_size,
           w_gru_update_gate_features, w_gru_forget_gate_features,
           w_gru_current_memory_message_features,
           u_gru_update_gate, u_gru_forget_gate, u_gru_current_memory_message,
           b_gru_update_gate, b_gru_forget_gate, b_gru_current_memory_message,
           u_graph_node_features, u_graph_neighbor_messages,
           linear_w, linear_b):
    del batch_size
    bz = b_gru_update_gate.reshape(1, _F)
    br = b_gru_forget_gate.reshape(1, _F)
    bm = b_gru_current_memory_message.reshape(1, _F)
    lb = linear_b.reshape(1, _FC_OUT)
    x_t = jnp.transpose(node_features, (1, 0, 2))
    a_t = jnp.transpose(adjacency_matrix, (1, 0, 2))
    lw_t = jnp.transpose(linear_w.reshape(_FC_OUT, _N, _F), (1, 0, 2))

    def _prep(w):
        return jnp.transpose(w, (0, 3, 2, 1)).astype(jnp.bfloat16)

    wz_p = _prep(w_gru_update_gate_features)
    wr_p = _prep(w_gru_forget_gate_features)
    wm_p = _prep(w_gru_current_memory_message_features)
    uz_p = _prep(u_gru_update_gate)
    ur_p = _prep(u_gru_forget_gate)
    um_p = _prep(u_gru_current_memory_message)

    grid = (_N // _BI,)
    wspec = pl.BlockSpec((_BI, _F, _F, _N), lambda i: (i, 0, 0, 0))
    sspec = pl.BlockSpec((_BI, _F, _F), lambda i: (i, 0, 0))
    full2 = lambda shape: pl.BlockSpec(shape, lambda i: (0, 0))

    return pl.pallas_call(
        _gnn_kernel,
        grid=grid,
        in_specs=[
            pl.BlockSpec((_BI, _B, _F), lambda i: (i, 0, 0)),
            pl.BlockSpec((_BI, _B, _N), lambda i: (i, 0, 0)),
            wspec, wspec, wspec, wspec, wspec, wspec,
            full2((1, _F)), full2((1, _F)), full2((1, _F)),
            sspec, sspec,
            pl.BlockSpec((_BI, _FC_OUT, _F), lambda i: (i, 0, 0)),
            full2((1, _FC_OUT)),
        ],
        out_specs=pl.BlockSpec((_B, _FC_OUT), lambda i: (0, 0)),
        out_shape=jax.ShapeDtypeStruct((_B, _FC_OUT), jnp.float32),
        compiler_params=pltpu.CompilerParams(
            dimension_semantics=("arbitrary",)),
    )(x_t, a_t, wz_p, wr_p, wm_p, uz_p, ur_p, um_p,
      bz, br, bm, u_graph_node_features, u_graph_neighbor_messages,
      lw_t, lb)
